# baseline (speedup 1.0000x reference)
import functools

import jax
import jax.numpy as jnp
from jax import lax
from jax.experimental import pallas as pl
from jax.experimental.pallas import tpu as pltpu
from jax.experimental.pallas import tpu_sc as plsc

N = 10000
E = 320000
D = 128

NC = 2
NS = 16
NW = NC * NS
EPW = E // NW
C = 128
BCH = 80
BCAP = BCH * C
NP = 10240
HALF = NP // NC
ACC = 5376
GARBAGE = HALF
ZPT = ACC // NS
CPT = HALF // NS

_mesh = plsc.VectorSubcoreMesh(
    core_axis_name="c", subcore_axis_name="s", num_cores=NC, num_subcores=NS)
_sc_params = pltpu.CompilerParams(needs_layout_passes=False)


@functools.partial(
    pl.kernel,
    out_type=(
        jax.ShapeDtypeStruct((NW, 2, N), jnp.float32),
        jax.ShapeDtypeStruct((NW, 2, BCAP), jnp.int32),
        jax.ShapeDtypeStruct((NW, 2, BCAP), jnp.int32),
        jax.ShapeDtypeStruct((NW, 2, 16), jnp.int32),
    ),
    mesh=_mesh,
    compiler_params=_sc_params,
    scratch_types=[
        pltpu.VMEM((2, EPW), jnp.int32),
        pltpu.VMEM((N,), jnp.float32),
        pltpu.VMEM((N,), jnp.float32),
        pltpu.VMEM((BCAP,), jnp.int32),
        pltpu.VMEM((BCAP,), jnp.int32),
        pltpu.VMEM((BCAP,), jnp.int32),
        pltpu.VMEM((BCAP,), jnp.int32),
        pltpu.VMEM((2, 16), jnp.int32),
    ],
)
def _deg_kernel(edges_hbm, cnt_hbm, srcb_hbm, dstb_hbm, bcnt_hbm,
                idx_v, cnt_s, cnt_d, srcb0, srcb1, dstb0, dstb1, bcnt):
    cid = lax.axis_index("c")
    sid = lax.axis_index("s")
    wid = sid * NC + cid
    pltpu.sync_copy(edges_hbm.at[0, wid], idx_v.at[0])
    pltpu.sync_copy(edges_hbm.at[1, wid], idx_v.at[1])

    zeros = jnp.zeros((16,), jnp.float32)
    zeros_i = jnp.zeros((16,), jnp.int32)
    garb = jnp.full((16,), GARBAGE, jnp.int32)

    def zbody(i, carry):
        cnt_s[pl.ds(i * 16, 16)] = zeros
        cnt_d[pl.ds(i * 16, 16)] = zeros
        return carry

    lax.fori_loop(0, N // 16, zbody, 0)

    def pf(i, carry):
        srcb0[pl.ds(i * 16, 16)] = zeros_i
        srcb1[pl.ds(i * 16, 16)] = zeros_i
        dstb0[pl.ds(i * 16, 16)] = garb
        dstb1[pl.ds(i * 16, 16)] = garb
        return carry

    lax.fori_loop(0, BCAP // 16, pf, 0)

    ones = jnp.ones((16,), jnp.float32)

    def body(i, carry):
        off0, off1 = carry
        s = idx_v[0, pl.ds(i * 16, 16)]
        d = idx_v[1, pl.ds(i * 16, 16)]
        plsc.addupdate_scatter(cnt_s, [s], ones)
        plsc.addupdate_scatter(cnt_d, [d], ones)
        m0 = d < HALF
        m1 = jnp.logical_not(m0)
        plsc.store_compressed(srcb0.at[pl.ds(off0, 16)], s, mask=m0)
        plsc.store_compressed(dstb0.at[pl.ds(off0, 16)], d, mask=m0)
        plsc.store_compressed(srcb1.at[pl.ds(off1, 16)], s, mask=m1)
        plsc.store_compressed(dstb1.at[pl.ds(off1, 16)], d - HALF, mask=m1)
        n0 = jnp.sum(m0.astype(jnp.int32))
        return off0 + n0, off1 + (16 - n0)

    off0, off1 = lax.fori_loop(0, EPW // 16, body, (0, 0))

    bcnt[0, :] = jnp.full((16,), (off0 + C - 1) // C, jnp.int32)
    bcnt[1, :] = jnp.full((16,), (off1 + C - 1) // C, jnp.int32)

    pltpu.sync_copy(cnt_s, cnt_hbm.at[wid, 0])
    pltpu.sync_copy(cnt_d, cnt_hbm.at[wid, 1])
    pltpu.sync_copy(srcb0, srcb_hbm.at[wid, 0])
    pltpu.sync_copy(srcb1, srcb_hbm.at[wid, 1])
    pltpu.sync_copy(dstb0, dstb_hbm.at[wid, 0])
    pltpu.sync_copy(dstb1, dstb_hbm.at[wid, 1])
    pltpu.sync_copy(bcnt.at[0], bcnt_hbm.at[wid, 0])
    pltpu.sync_copy(bcnt.at[1], bcnt_hbm.at[wid, 1])


@functools.partial(
    pl.kernel,
    out_type=jax.ShapeDtypeStruct((NP, D), jnp.float32),
    mesh=_mesh,
    compiler_params=_sc_params,
    scratch_types=[
        pltpu.VMEM((2, BCH, C), jnp.int32),
        pltpu.VMEM((2, BCH, C), jnp.int32),
        pltpu.VMEM((2, 16), jnp.int32),
        pltpu.VMEM((C, D), jnp.float32),
        pltpu.VMEM((C, D), jnp.float32),
        pltpu.VMEM_SHARED((ACC, D), jnp.float32),
        pltpu.SemaphoreType.DMA,
        pltpu.SemaphoreType.DMA,
    ],
)
def _agg_kernel(u_hbm, srcb_hbm, dstb_hbm, bcnt_hbm, out_hbm,
                sidx, didx, bcnt, rows, rows1, acc, sem, sem1):
    cid = lax.axis_index("c")
    sid = lax.axis_index("s")

    for k in range(2):
        w = 2 * sid + k
        pltpu.sync_copy(srcb_hbm.at[w, cid], sidx.at[k])
        pltpu.sync_copy(dstb_hbm.at[w, cid], didx.at[k])
        pltpu.sync_copy(bcnt_hbm.at[w, cid], bcnt.at[k])

    zeros = jnp.zeros((16,), jnp.float32)

    def zb(i, carry):
        rows[i // 8, pl.ds((i % 8) * 16, 16)] = zeros
        return carry

    lax.fori_loop(0, C * (D // 16), zb, 0)
    pltpu.sync_copy(rows, acc.at[pl.ds(sid * ZPT, C)])
    pltpu.sync_copy(rows, acc.at[pl.ds(sid * ZPT + C, C)])
    pltpu.sync_copy(rows.at[pl.ds(0, ZPT - 2 * C)],
                    acc.at[pl.ds(sid * ZPT + 2 * C, ZPT - 2 * C)])
    plsc.subcore_barrier()

    for k in range(2):
        nch = jnp.max(bcnt[k, :])
        nch_e = jnp.maximum(((nch + 1) // 2) * 2, 2)

        def body(g, carry):
            i0 = 2 * g
            c0 = pltpu.async_copy(u_hbm.at[sidx.at[k, i0]], rows, sem)
            c1 = pltpu.async_copy(u_hbm.at[sidx.at[k, i0 + 1]], rows1, sem1)
            c0.wait()
            pltpu.sync_copy(rows, acc.at[didx.at[k, i0]], add=True)
            c1.wait()
            pltpu.sync_copy(rows1, acc.at[didx.at[k, i0 + 1]], add=True)
            return carry

        lax.fori_loop(0, nch_e // 2, body, 0)

    plsc.subcore_barrier()
    pltpu.sync_copy(
        acc.at[pl.ds(sid * CPT, CPT)],
        out_hbm.at[pl.ds(cid * HALF + sid * CPT, CPT)],
    )


def _a0_body(cnt_ref, rr_ref):
    deg = jnp.sum(cnt_ref[...], axis=0)
    rr_ref[...] = lax.rsqrt(jnp.maximum(deg, 1.0))


def _a1_body(x_ref, routc_ref, w1_ref, u_ref):
    u_ref[...] = jnp.dot(
        x_ref[...] * routc_ref[...], w1_ref[...], preferred_element_type=jnp.float32
    )


def _b_body(p_ref, rinc_ref, routc_ref, b1_ref, w2_ref, v_ref):
    agg = p_ref[...] * rinc_ref[...]
    y1 = jnp.tanh(agg + b1_ref[...])
    v_ref[...] = jnp.dot(
        y1 * routc_ref[...], w2_ref[...], preferred_element_type=jnp.float32
    )


def _c_body(p_ref, rinc_ref, b2_ref, o_ref):
    t = p_ref[...] * rinc_ref[...] + b2_ref[...]
    o_ref[...] = jnp.concatenate([t[:, D // 2:], -t[:, : D // 2]], axis=1)


_a0_call = pl.pallas_call(
    _a0_body, out_shape=jax.ShapeDtypeStruct((2, N), jnp.float32))
_a1_call = pl.pallas_call(
    _a1_body, out_shape=jax.ShapeDtypeStruct((N, D), jnp.float32))
_b_call = pl.pallas_call(
    _b_body, out_shape=jax.ShapeDtypeStruct((N, D), jnp.float32))
_c_call = pl.pallas_call(
    _c_body, out_shape=jax.ShapeDtypeStruct((N, D), jnp.float32))


def kernel(x, edge_index, W1, b1, W2, b2):
    edges2 = edge_index.reshape(2, NW, EPW)

    cnts, srcb, dstb, bcnt = _deg_kernel(edges2)
    rr = _a0_call(cnts)
    routc = rr[0].reshape(N, 1)
    rinc = rr[1].reshape(N, 1)

    srcb4 = srcb.reshape(NW, 2, BCH, C)
    dstb4 = dstb.reshape(NW, 2, BCH, C)

    u = _a1_call(x, routc, W1)
    p1 = _agg_kernel(u, srcb4, dstb4, bcnt)[:N]
    v = _b_call(p1, rinc, routc, b1.reshape(1, D), W2)
    p2 = _agg_kernel(v, srcb4, dstb4, bcnt)[:N]
    out = _c_call(p2, rinc, b2.reshape(1, D))
    return out


# speedup vs baseline: 1.3431x; 1.3431x over previous
import functools

import jax
import jax.numpy as jnp
from jax import lax
from jax.experimental import pallas as pl
from jax.experimental.pallas import tpu as pltpu
from jax.experimental.pallas import tpu_sc as plsc

N = 10000
E = 320000
D = 128

NC = 2
NS = 16
NW = NC * NS
EPW = E // NW
C = 128
BCH = 79
BCAP = BCH * C
NP = 10240
HALF = NP // NC
ACC = 5376
GARBAGE = HALF
ZPT = ACC // NS
CPT = HALF // NS

_mesh = plsc.VectorSubcoreMesh(
    core_axis_name="c", subcore_axis_name="s", num_cores=NC, num_subcores=NS)
_sc_params = pltpu.CompilerParams(needs_layout_passes=False)


@functools.partial(
    pl.kernel,
    out_type=(
        jax.ShapeDtypeStruct((NW, 2, N), jnp.float32),
        jax.ShapeDtypeStruct((NW, 2, BCAP), jnp.int32),
        jax.ShapeDtypeStruct((NW, 2, BCAP), jnp.int32),
        jax.ShapeDtypeStruct((NW, 2, 16), jnp.int32),
    ),
    mesh=_mesh,
    compiler_params=_sc_params,
    scratch_types=[
        pltpu.VMEM((2, EPW), jnp.int32),
        pltpu.VMEM((N,), jnp.float32),
        pltpu.VMEM((N,), jnp.float32),
        pltpu.VMEM((BCAP,), jnp.int32),
        pltpu.VMEM((BCAP,), jnp.int32),
        pltpu.VMEM((BCAP,), jnp.int32),
        pltpu.VMEM((BCAP,), jnp.int32),
        pltpu.VMEM((2, 16), jnp.int32),
    ],
)
def _deg_kernel(edges_hbm, cnt_hbm, srcb_hbm, dstb_hbm, bcnt_hbm,
                idx_v, cnt_s, cnt_d, srcb0, srcb1, dstb0, dstb1, bcnt):
    cid = lax.axis_index("c")
    sid = lax.axis_index("s")
    wid = sid * NC + cid
    pltpu.sync_copy(edges_hbm.at[0, wid], idx_v.at[0])
    pltpu.sync_copy(edges_hbm.at[1, wid], idx_v.at[1])

    zeros = jnp.zeros((16,), jnp.float32)
    zeros_i = jnp.zeros((16,), jnp.int32)
    garb = jnp.full((16,), GARBAGE, jnp.int32)

    def zbody(i, carry):
        cnt_s[pl.ds(i * 16, 16)] = zeros
        cnt_d[pl.ds(i * 16, 16)] = zeros
        return carry

    lax.fori_loop(0, N // 16, zbody, 0)

    def pf(i, carry):
        srcb0[pl.ds(i * 16, 16)] = zeros_i
        srcb1[pl.ds(i * 16, 16)] = zeros_i
        dstb0[pl.ds(i * 16, 16)] = garb
        dstb1[pl.ds(i * 16, 16)] = garb
        return carry

    lax.fori_loop(0, BCAP // 16, pf, 0)

    ones = jnp.ones((16,), jnp.float32)

    def body(i, carry):
        off0, off1 = carry
        s = idx_v[0, pl.ds(i * 16, 16)]
        d = idx_v[1, pl.ds(i * 16, 16)]
        plsc.addupdate_scatter(cnt_s, [s], ones)
        plsc.addupdate_scatter(cnt_d, [d], ones)
        m0 = d < HALF
        m1 = jnp.logical_not(m0)
        plsc.store_compressed(srcb0.at[pl.ds(off0, 16)], s, mask=m0)
        plsc.store_compressed(dstb0.at[pl.ds(off0, 16)], d, mask=m0)
        plsc.store_compressed(srcb1.at[pl.ds(off1, 16)], s, mask=m1)
        plsc.store_compressed(dstb1.at[pl.ds(off1, 16)], d - HALF, mask=m1)
        n0 = jnp.sum(m0.astype(jnp.int32))
        return off0 + n0, off1 + (16 - n0)

    off0, off1 = lax.fori_loop(0, EPW // 16, body, (0, 0))

    bcnt[0, :] = jnp.full((16,), (off0 + C - 1) // C, jnp.int32)
    bcnt[1, :] = jnp.full((16,), (off1 + C - 1) // C, jnp.int32)

    pltpu.sync_copy(cnt_s, cnt_hbm.at[wid, 0])
    pltpu.sync_copy(cnt_d, cnt_hbm.at[wid, 1])
    pltpu.sync_copy(srcb0, srcb_hbm.at[wid, 0])
    pltpu.sync_copy(srcb1, srcb_hbm.at[wid, 1])
    pltpu.sync_copy(dstb0, dstb_hbm.at[wid, 0])
    pltpu.sync_copy(dstb1, dstb_hbm.at[wid, 1])
    pltpu.sync_copy(bcnt.at[0], bcnt_hbm.at[wid, 0])
    pltpu.sync_copy(bcnt.at[1], bcnt_hbm.at[wid, 1])


@functools.partial(
    pl.kernel,
    out_type=jax.ShapeDtypeStruct((NP, D), jnp.float32),
    mesh=_mesh,
    compiler_params=_sc_params,
    scratch_types=[
        pltpu.VMEM((2, BCH, C), jnp.int32),
        pltpu.VMEM((2, BCH, C), jnp.int32),
        pltpu.VMEM((2, 16), jnp.int32),
        pltpu.VMEM((C, D), jnp.float32),
        pltpu.VMEM_SHARED((ACC, D), jnp.float32),
        pltpu.SemaphoreType.DMA,
    ],
)
def _agg_kernel(u_hbm, srcb_hbm, dstb_hbm, bcnt_hbm, out_hbm,
                sidx, didx, bcnt, rows, acc, sem):
    cid = lax.axis_index("c")
    sid = lax.axis_index("s")

    for k in range(2):
        w = 2 * sid + k
        pltpu.sync_copy(srcb_hbm.at[w, cid], sidx.at[k])
        pltpu.sync_copy(dstb_hbm.at[w, cid], didx.at[k])
        pltpu.sync_copy(bcnt_hbm.at[w, cid], bcnt.at[k])

    zeros = jnp.zeros((16,), jnp.float32)

    def zb(i, carry):
        rows[i // 8, pl.ds((i % 8) * 16, 16)] = zeros
        return carry

    lax.fori_loop(0, C * (D // 16), zb, 0)
    pltpu.sync_copy(rows, acc.at[pl.ds(sid * ZPT, C)])
    pltpu.sync_copy(rows, acc.at[pl.ds(sid * ZPT + C, C)])
    pltpu.sync_copy(rows.at[pl.ds(0, ZPT - 2 * C)],
                    acc.at[pl.ds(sid * ZPT + 2 * C, ZPT - 2 * C)])
    plsc.subcore_barrier()

    for k in range(2):
        nch = jnp.max(bcnt[k, :])

        def body(i, carry):
            pltpu.sync_copy(u_hbm.at[sidx.at[k, i]], rows)
            pltpu.sync_copy(rows, acc.at[didx.at[k, i]], add=True)
            return carry

        lax.fori_loop(0, nch, body, 0)

    plsc.subcore_barrier()
    pltpu.sync_copy(
        acc.at[pl.ds(sid * CPT, CPT)],
        out_hbm.at[pl.ds(cid * HALF + sid * CPT, CPT)],
    )


def _a0_body(cnt_ref, rr_ref):
    deg = jnp.sum(cnt_ref[...], axis=0)
    rr_ref[...] = lax.rsqrt(jnp.maximum(deg, 1.0))


def _a1_body(x_ref, routc_ref, w1_ref, u_ref):
    u_ref[...] = jnp.dot(
        x_ref[...] * routc_ref[...], w1_ref[...], preferred_element_type=jnp.float32
    )


def _b_body(p_ref, rinc_ref, routc_ref, b1_ref, w2_ref, v_ref):
    agg = p_ref[...] * rinc_ref[...]
    y1 = jnp.tanh(agg + b1_ref[...])
    v_ref[...] = jnp.dot(
        y1 * routc_ref[...], w2_ref[...], preferred_element_type=jnp.float32
    )


def _c_body(p_ref, rinc_ref, b2_ref, o_ref):
    t = p_ref[...] * rinc_ref[...] + b2_ref[...]
    o_ref[...] = jnp.concatenate([t[:, D // 2:], -t[:, : D // 2]], axis=1)


_a0_call = pl.pallas_call(
    _a0_body, out_shape=jax.ShapeDtypeStruct((2, N), jnp.float32))
_a1_call = pl.pallas_call(
    _a1_body, out_shape=jax.ShapeDtypeStruct((N, D), jnp.float32))
_b_call = pl.pallas_call(
    _b_body, out_shape=jax.ShapeDtypeStruct((N, D), jnp.float32))
_c_call = pl.pallas_call(
    _c_body, out_shape=jax.ShapeDtypeStruct((N, D), jnp.float32))


def kernel(x, edge_index, W1, b1, W2, b2):
    edges2 = edge_index.reshape(2, NW, EPW)

    cnts, srcb, dstb, bcnt = _deg_kernel(edges2)
    rr = _a0_call(cnts)
    routc = rr[0].reshape(N, 1)
    rinc = rr[1].reshape(N, 1)

    srcb4 = srcb.reshape(NW, 2, BCH, C)
    dstb4 = dstb.reshape(NW, 2, BCH, C)

    u = _a1_call(x, routc, W1)
    p1 = _agg_kernel(u, srcb4, dstb4, bcnt)[:N]
    v = _b_call(p1, rinc, routc, b1.reshape(1, D), W2)
    p2 = _agg_kernel(v, srcb4, dstb4, bcnt)[:N]
    out = _c_call(p2, rinc, b2.reshape(1, D))
    return out
